# baseline (speedup 1.0000x reference)
import functools
import math

import jax
import jax.numpy as jnp
from jax import lax
from jax.experimental import pallas as pl
from jax.experimental.pallas import tpu as pltpu
from jax.experimental.pallas import tpu_sc as plsc

D = 128
H = 4
C = 32
N_CELLS = 2000
N_GENES = 64
E_SPATIAL = 32000
E_GRN = 256

CB = 8
R = CB * N_GENES

NC = 2
NS = 16
NW = NC * NS
EPW = E_SPATIAL // NW
CHUNK = 128
NFULL = EPW // CHUNK
REM = EPW - NFULL * CHUNK
ACC_ROWS = 2048
OWN = ACC_ROWS // NS


def _sc_agg_body(x_hbm, src_hbm, dst_hbm, out_hbm,
                 rows_v, rows_rem_v, src_v, src_rem_v, dst_v, dst_rem_v,
                 zero_v, core_acc, sem):
    c = lax.axis_index("c")
    s = lax.axis_index("s")
    w = s * NC + c

    def _zr(i, _):
        for j in range(D // 16):
            zero_v[i, pl.ds(j * 16, 16)] = jnp.zeros((16,), jnp.float32)
        return 0
    lax.fori_loop(0, OWN, _zr, 0)

    pltpu.sync_copy(zero_v, core_acc.at[pl.ds(s * OWN, OWN)])
    plsc.subcore_barrier()

    base = w * EPW
    for j in range(NFULL):
        off = base + j * CHUNK
        pltpu.sync_copy(src_hbm.at[pl.ds(off, CHUNK)], src_v.at[0])
        pltpu.sync_copy(dst_hbm.at[pl.ds(off, CHUNK)], dst_v.at[0])
        pltpu.async_copy(x_hbm.at[src_v.at[0]], rows_v, sem).wait()
        pltpu.sync_copy(rows_v, core_acc.at[dst_v.at[0]], add=True)
    off = base + NFULL * CHUNK
    pltpu.sync_copy(src_hbm.at[pl.ds(off, REM)], src_rem_v.at[0])
    pltpu.sync_copy(dst_hbm.at[pl.ds(off, REM)], dst_rem_v.at[0])
    pltpu.async_copy(x_hbm.at[src_rem_v.at[0]], rows_rem_v, sem).wait()
    pltpu.sync_copy(rows_rem_v, core_acc.at[dst_rem_v.at[0]], add=True)

    plsc.subcore_barrier()
    pltpu.sync_copy(core_acc.at[pl.ds(s * OWN, OWN)], zero_v)
    pltpu.sync_copy(zero_v, out_hbm.at[c, pl.ds(s * OWN, OWN)])


def _sc_agg(high_emb, spatial_edge_index):
    mesh = plsc.VectorSubcoreMesh(core_axis_name="c", subcore_axis_name="s")
    k = functools.partial(
        pl.kernel,
        mesh=mesh,
        out_type=jax.ShapeDtypeStruct((NC, ACC_ROWS, D), jnp.float32),
        scratch_types=[
            pltpu.VMEM((CHUNK, D), jnp.float32),
            pltpu.VMEM((REM, D), jnp.float32),
            pltpu.VMEM((1, CHUNK), jnp.int32),
            pltpu.VMEM((1, REM), jnp.int32),
            pltpu.VMEM((1, CHUNK), jnp.int32),
            pltpu.VMEM((1, REM), jnp.int32),
            pltpu.VMEM((OWN, D), jnp.float32),
            pltpu.VMEM_SHARED((ACC_ROWS, D), jnp.float32),
            pltpu.SemaphoreType.DMA,
        ],
    )(_sc_agg_body)
    return k(high_emb, spatial_edge_index[0], spatial_edge_index[1])


def _layer_norm(x, g, b):
    m = jnp.mean(x, axis=-1, keepdims=True)
    v = jnp.mean((x - m) ** 2, axis=-1, keepdims=True)
    return (x - m) * lax.rsqrt(v + 1e-5) * g + b


def _gelu(x):
    return x * 0.5 * (1.0 + lax.erf(x * (2.0 ** -0.5)))


def _a_body(high_ref, agg_ref, grn_ref, w1_ref, b1_ref, w2_ref, b2_ref,
            eps_ref, g_ref, beta_ref, ho_ref, mt_ref):
    x = high_ref[...]
    agg = agg_ref[0, :N_CELLS, :] + agg_ref[1, :N_CELLS, :]
    h = (1.0 + eps_ref[0, 0]) * x + agg
    h = _gelu(jnp.dot(h, w1_ref[...], preferred_element_type=jnp.float32) + b1_ref[...])
    h = jnp.dot(h, w2_ref[...], preferred_element_type=jnp.float32) + b2_ref[...]
    ho_ref[...] = _layer_norm(h, g_ref[...], beta_ref[...])

    src = grn_ref[0:1, :]
    dst = grn_ref[1:2, :]
    gi = lax.broadcasted_iota(jnp.int32, (N_GENES, E_GRN), 0)
    dst_oh = (gi == dst).astype(jnp.float32)
    src_oh = (gi == src).astype(jnp.float32)
    m = lax.dot_general(dst_oh, src_oh, (((1,), (1,)), ((), ())),
                        preferred_element_type=jnp.float32)
    mrep = jnp.concatenate([jnp.concatenate([m] * CB, axis=1)] * CB, axis=0)
    rc = lax.broadcasted_iota(jnp.int32, (R, R), 0) // N_GENES
    cc = lax.broadcasted_iota(jnp.int32, (R, R), 1) // N_GENES
    mt_ref[...] = jnp.where(rc == cc, mrep, 0.0)


def _b_body(low_ref, he_ref, ho_ref, mt_ref, wqkvs_ref, bqkvs_ref,
            tcg_ref, tcb_ref, aggw_ref, aggb_ref,
            h2lwq_ref, h2lbq_ref, h2lwkv_ref, h2lbkv_ref,
            l2hwq_ref, l2hbq_ref, l2hwkv_ref, l2hbkv_ref,
            nhg_ref, nhb_ref, nlg_ref, nlb_ref,
            hn_ref, ln_ref):
    x = low_ref[...].reshape(R, D)
    qkvs = jnp.dot(x.astype(jnp.bfloat16), wqkvs_ref[...],
                   preferred_element_type=jnp.float32) + bqkvs_ref[...]
    q = qkvs[:, 0:D]
    k = qkvs[:, D:2 * D]
    v = qkvs[:, 2 * D:3 * D]
    skip = qkvs[:, 3 * D:4 * D]
    mt = mt_ref[...]

    outs = []
    for h in range(H):
        qh = q[:, h * C:(h + 1) * C].astype(jnp.bfloat16)
        kh = k[:, h * C:(h + 1) * C].astype(jnp.bfloat16)
        vh = v[:, h * C:(h + 1) * C].astype(jnp.bfloat16)
        logit = lax.dot_general(qh, kh, (((1,), (1,)), ((), ())),
                                preferred_element_type=jnp.float32)
        ex = mt * jnp.exp(logit)
        den = jnp.sum(ex, axis=1, keepdims=True)
        den = jnp.where(den > 0.0, den, 1.0)
        oh = jnp.dot(ex.astype(jnp.bfloat16), vh,
                     preferred_element_type=jnp.float32) / den
        outs.append(oh)
    attn = jnp.concatenate(outs, axis=1) + skip
    low_out = _layer_norm(attn, tcg_ref[...], tcb_ref[...])

    ii = lax.broadcasted_iota(jnp.int32, (CB, R), 0)
    jj = lax.broadcasted_iota(jnp.int32, (CB, R), 1)
    pool = jnp.where(jj // N_GENES == ii, 1.0 / N_GENES, 0.0).astype(jnp.bfloat16)
    gmean = jnp.dot(pool, low_out.astype(jnp.bfloat16),
                    preferred_element_type=jnp.float32)
    gene = _gelu(jnp.dot(gmean, aggw_ref[...], preferred_element_type=jnp.float32) + aggb_ref[...])

    ho = ho_ref[0]
    he = he_ref[0]
    dscale = 1.0 / math.sqrt(float(D))
    qh2l = jnp.dot(ho, h2lwq_ref[...], preferred_element_type=jnp.float32) + h2lbq_ref[...]
    kvh = jnp.dot(gene, h2lwkv_ref[...], preferred_element_type=jnp.float32) + h2lbkv_ref[...]
    ah = jax.nn.sigmoid(jnp.sum(qh2l * kvh[:, 0:D], axis=1, keepdims=True) * dscale)
    high_cross = ah * kvh[:, D:2 * D]

    ql2h = jnp.dot(gene, l2hwq_ref[...], preferred_element_type=jnp.float32) + l2hbq_ref[...]
    kvl = jnp.dot(ho, l2hwkv_ref[...], preferred_element_type=jnp.float32) + l2hbkv_ref[...]
    al = jax.nn.sigmoid(jnp.sum(ql2h * kvl[:, 0:D], axis=1, keepdims=True) * dscale)
    low_cross = al * kvl[:, D:2 * D]

    hn_ref[...] = _layer_norm(he + ho + high_cross, nhg_ref[...], nhb_ref[...]).reshape(1, CB, D)

    ri = lax.broadcasted_iota(jnp.int32, (R, CB), 0)
    rj = lax.broadcasted_iota(jnp.int32, (R, CB), 1)
    expand = jnp.where(ri // N_GENES == rj, 1.0, 0.0).astype(jnp.bfloat16)
    z = x + low_out + jnp.dot(expand, low_cross.astype(jnp.bfloat16),
                              preferred_element_type=jnp.float32)
    ln_ref[...] = _layer_norm(z, nlg_ref[...], nlb_ref[...]).reshape(CB, N_GENES, D)


def _full(shape):
    return pl.BlockSpec(shape, lambda i: tuple(0 for _ in shape))


_A_KWARGS = dict(
    grid=(1,),
    in_specs=[
        pl.BlockSpec((N_CELLS, D), lambda i: (0, 0)),
        pl.BlockSpec((NC, ACC_ROWS, D), lambda i: (0, 0, 0)),
        pl.BlockSpec((2, E_GRN), lambda i: (0, 0)),
        _full((D, 2 * D)), _full((1, 2 * D)),
        _full((2 * D, D)), _full((1, D)),
        _full((1, 1)), _full((1, D)), _full((1, D)),
    ],
    out_specs=[
        pl.BlockSpec((N_CELLS, D), lambda i: (0, 0)),
        pl.BlockSpec((R, R), lambda i: (0, 0)),
    ],
    out_shape=[
        jax.ShapeDtypeStruct((N_CELLS, D), jnp.float32),
        jax.ShapeDtypeStruct((R, R), jnp.float32),
    ],
)

_B_KWARGS = dict(
    grid=(N_CELLS // CB,),
    in_specs=[
        pl.BlockSpec((CB, N_GENES, D), lambda i: (i, 0, 0)),
        pl.BlockSpec((1, CB, D), lambda i: (i, 0, 0)),
        pl.BlockSpec((1, CB, D), lambda i: (i, 0, 0)),
        _full((R, R)),
        _full((D, 4 * D)), _full((1, 4 * D)),
        _full((1, D)), _full((1, D)),
        _full((D, D)), _full((1, D)),
        _full((D, D)), _full((1, D)),
        _full((D, 2 * D)), _full((1, 2 * D)),
        _full((D, D)), _full((1, D)),
        _full((D, 2 * D)), _full((1, 2 * D)),
        _full((1, D)), _full((1, D)), _full((1, D)), _full((1, D)),
    ],
    out_specs=[
        pl.BlockSpec((1, CB, D), lambda i: (i, 0, 0)),
        pl.BlockSpec((CB, N_GENES, D), lambda i: (i, 0, 0)),
    ],
    out_shape=[
        jax.ShapeDtypeStruct((N_CELLS // CB, CB, D), jnp.float32),
        jax.ShapeDtypeStruct((N_CELLS, N_GENES, D), jnp.float32),
    ],
)


def _row(b):
    return b.reshape(1, -1)


def kernel(high_emb, low_emb, spatial_edge_index, grn_edge_index, params):
    p = params
    agg_parts = _sc_agg(high_emb, spatial_edge_index.astype(jnp.int32))

    high_out, mtile = pl.pallas_call(_a_body, **_A_KWARGS)(
        high_emb, agg_parts, grn_edge_index.astype(jnp.int32),
        p["gin_W1"], _row(p["gin_b1"]), p["gin_W2"], _row(p["gin_b2"]),
        p["gin_eps"].reshape(1, 1), _row(p["gin_ln_g"]), _row(p["gin_ln_b"]),
    )

    s = 1.0 / math.sqrt(float(C))
    wqkvs = jnp.concatenate([p["tc_Wq"] * s, p["tc_Wk"], p["tc_Wv"], p["tc_Wskip"]],
                            axis=1).astype(jnp.bfloat16)
    bqkvs = jnp.concatenate([p["tc_bq"] * s, p["tc_bk"], p["tc_bv"], p["tc_bskip"]]).reshape(1, -1)
    h2lwkv = jnp.concatenate([p["h2l_Wk"], p["h2l_Wv"]], axis=1)
    h2lbkv = jnp.concatenate([p["h2l_bk"], p["h2l_bv"]]).reshape(1, -1)
    l2hwkv = jnp.concatenate([p["l2h_Wk"], p["l2h_Wv"]], axis=1)
    l2hbkv = jnp.concatenate([p["l2h_bk"], p["l2h_bv"]]).reshape(1, -1)

    high_new, low_new = pl.pallas_call(_b_body, **_B_KWARGS)(
        low_emb, high_emb.reshape(N_CELLS // CB, CB, D),
        high_out.reshape(N_CELLS // CB, CB, D), mtile,
        wqkvs, bqkvs, _row(p["tc_ln_g"]), _row(p["tc_ln_b"]),
        p["agg_W"], _row(p["agg_b"]),
        p["h2l_Wq"], _row(p["h2l_bq"]), h2lwkv, h2lbkv,
        p["l2h_Wq"], _row(p["l2h_bq"]), l2hwkv, l2hbkv,
        _row(p["nh_g"]), _row(p["nh_b"]), _row(p["nl_g"]), _row(p["nl_b"]),
    )
    return (high_new.reshape(N_CELLS, D), low_new)


# speedup vs baseline: 1.2274x; 1.2274x over previous
import functools
import math

import jax
import jax.numpy as jnp
from jax import lax
from jax.experimental import pallas as pl
from jax.experimental.pallas import tpu as pltpu
from jax.experimental.pallas import tpu_sc as plsc

D = 128
H = 4
C = 32
N_CELLS = 2000
N_GENES = 64
E_SPATIAL = 32000
E_GRN = 256

CB = 8
R = CB * N_GENES

NC = 2
NS = 16
NW = NC * NS
EPW = E_SPATIAL // NW
CHUNK = 128
NFULL = EPW // CHUNK
REM = EPW - NFULL * CHUNK
ACC_ROWS = 2048
OWN = ACC_ROWS // NS


def _sc_agg_body(x_hbm, src_hbm, dst_hbm, out_hbm,
                 rows_v, rows_rem_v, src_v, src_rem_v, dst_v, dst_rem_v,
                 zero_v, core_acc, sem):
    c = lax.axis_index("c")
    s = lax.axis_index("s")
    w = s * NC + c

    def _zr(i, _):
        for j in range(D // 16):
            zero_v[i, pl.ds(j * 16, 16)] = jnp.zeros((16,), jnp.float32)
        return 0
    lax.fori_loop(0, OWN, _zr, 0)

    pltpu.sync_copy(zero_v, core_acc.at[pl.ds(s * OWN, OWN)])
    plsc.subcore_barrier()

    base = w * EPW
    for j in range(NFULL):
        off = base + j * CHUNK
        pltpu.sync_copy(src_hbm.at[pl.ds(off, CHUNK)], src_v.at[0])
        pltpu.sync_copy(dst_hbm.at[pl.ds(off, CHUNK)], dst_v.at[0])
        pltpu.async_copy(x_hbm.at[src_v.at[0]], rows_v, sem).wait()
        pltpu.sync_copy(rows_v, core_acc.at[dst_v.at[0]], add=True)
    off = base + NFULL * CHUNK
    pltpu.sync_copy(src_hbm.at[pl.ds(off, REM)], src_rem_v.at[0])
    pltpu.sync_copy(dst_hbm.at[pl.ds(off, REM)], dst_rem_v.at[0])
    pltpu.async_copy(x_hbm.at[src_rem_v.at[0]], rows_rem_v, sem).wait()
    pltpu.sync_copy(rows_rem_v, core_acc.at[dst_rem_v.at[0]], add=True)

    plsc.subcore_barrier()
    pltpu.sync_copy(core_acc.at[pl.ds(s * OWN, OWN)], zero_v)
    pltpu.sync_copy(zero_v, out_hbm.at[c, pl.ds(s * OWN, OWN)])


def _sc_agg(high_emb, spatial_edge_index):
    mesh = plsc.VectorSubcoreMesh(core_axis_name="c", subcore_axis_name="s")
    k = functools.partial(
        pl.kernel,
        mesh=mesh,
        out_type=jax.ShapeDtypeStruct((NC, ACC_ROWS, D), jnp.float32),
        scratch_types=[
            pltpu.VMEM((CHUNK, D), jnp.float32),
            pltpu.VMEM((REM, D), jnp.float32),
            pltpu.VMEM((1, CHUNK), jnp.int32),
            pltpu.VMEM((1, REM), jnp.int32),
            pltpu.VMEM((1, CHUNK), jnp.int32),
            pltpu.VMEM((1, REM), jnp.int32),
            pltpu.VMEM((OWN, D), jnp.float32),
            pltpu.VMEM_SHARED((ACC_ROWS, D), jnp.float32),
            pltpu.SemaphoreType.DMA,
        ],
    )(_sc_agg_body)
    return k(high_emb, spatial_edge_index[0], spatial_edge_index[1])


def _layer_norm(x, g, b):
    m = jnp.mean(x, axis=-1, keepdims=True)
    v = jnp.mean((x - m) ** 2, axis=-1, keepdims=True)
    return (x - m) * lax.rsqrt(v + 1e-5) * g + b


def _gelu(x):
    return x * 0.5 * (1.0 + lax.erf(x * (2.0 ** -0.5)))


def _a_body(high_ref, agg_ref, grn_ref, w1_ref, b1_ref, w2_ref, b2_ref,
            eps_ref, g_ref, beta_ref, ho_ref, mt_ref):
    x = high_ref[...]
    agg = agg_ref[0, :N_CELLS, :] + agg_ref[1, :N_CELLS, :]
    h = (1.0 + eps_ref[0, 0]) * x + agg
    h = _gelu(jnp.dot(h, w1_ref[...], preferred_element_type=jnp.float32) + b1_ref[...])
    h = jnp.dot(h, w2_ref[...], preferred_element_type=jnp.float32) + b2_ref[...]
    ho_ref[...] = _layer_norm(h, g_ref[...], beta_ref[...])

    src = grn_ref[0:1, :]
    dst = grn_ref[1:2, :]
    gi = lax.broadcasted_iota(jnp.int32, (N_GENES, E_GRN), 0)
    dst_oh = (gi == dst).astype(jnp.float32)
    src_oh = (gi == src).astype(jnp.float32)
    m = lax.dot_general(dst_oh, src_oh, (((1,), (1,)), ((), ())),
                        preferred_element_type=jnp.float32)
    mrep = jnp.concatenate([jnp.concatenate([m] * CB, axis=1)] * CB, axis=0)
    rc = lax.broadcasted_iota(jnp.int32, (R, R), 0) // N_GENES
    cc = lax.broadcasted_iota(jnp.int32, (R, R), 1) // N_GENES
    mt = jnp.where(rc == cc, mrep, 0.0)
    mt_ref[...] = jnp.where(mt > 0.0, jnp.log(jnp.maximum(mt, 1e-30)), -1e30)


def _b_body(low_ref, he_ref, ho_ref, mt_ref, wqkvs_ref, bqkvs_ref,
            tcg_ref, tcb_ref, aggw_ref, aggb_ref,
            h2lwq_ref, h2lbq_ref, h2lwkv_ref, h2lbkv_ref,
            l2hwq_ref, l2hbq_ref, l2hwkv_ref, l2hbkv_ref,
            nhg_ref, nhb_ref, nlg_ref, nlb_ref,
            hn_ref, ln_ref):
    x = low_ref[...].reshape(R, D)
    qkvs = jnp.dot(x.astype(jnp.bfloat16), wqkvs_ref[...],
                   preferred_element_type=jnp.float32) + bqkvs_ref[...]
    q = qkvs[:, 0:D]
    k = qkvs[:, D:2 * D]
    v = qkvs[:, 2 * D:3 * D]
    skip = qkvs[:, 3 * D:4 * D]
    mt = mt_ref[...]

    ones_col = jnp.ones((R, 1), jnp.float32)
    outs = []
    for h in range(H):
        qh = q[:, h * C:(h + 1) * C].astype(jnp.bfloat16)
        kh = k[:, h * C:(h + 1) * C].astype(jnp.bfloat16)
        vh1 = jnp.concatenate([v[:, h * C:(h + 1) * C], ones_col], axis=1)
        logit = lax.dot_general(qh, kh, (((1,), (1,)), ((), ())),
                                preferred_element_type=jnp.float32)
        ex = jnp.exp(logit + mt)
        od = jnp.dot(ex, vh1, preferred_element_type=jnp.float32)
        den = od[:, C:C + 1]
        den = jnp.where(den > 0.0, den, 1.0)
        outs.append(od[:, 0:C] / den)
    attn = jnp.concatenate(outs, axis=1) + skip
    low_out = _layer_norm(attn, tcg_ref[...], tcb_ref[...])

    ii = lax.broadcasted_iota(jnp.int32, (CB, R), 0)
    jj = lax.broadcasted_iota(jnp.int32, (CB, R), 1)
    pool = jnp.where(jj // N_GENES == ii, 1.0 / N_GENES, 0.0).astype(jnp.bfloat16)
    gmean = jnp.dot(pool, low_out.astype(jnp.bfloat16),
                    preferred_element_type=jnp.float32)
    gene = _gelu(jnp.dot(gmean, aggw_ref[...], preferred_element_type=jnp.float32) + aggb_ref[...])

    ho = ho_ref[0]
    he = he_ref[0]
    dscale = 1.0 / math.sqrt(float(D))
    qh2l = jnp.dot(ho, h2lwq_ref[...], preferred_element_type=jnp.float32) + h2lbq_ref[...]
    kvh = jnp.dot(gene, h2lwkv_ref[...], preferred_element_type=jnp.float32) + h2lbkv_ref[...]
    ah = jax.nn.sigmoid(jnp.sum(qh2l * kvh[:, 0:D], axis=1, keepdims=True) * dscale)
    high_cross = ah * kvh[:, D:2 * D]

    ql2h = jnp.dot(gene, l2hwq_ref[...], preferred_element_type=jnp.float32) + l2hbq_ref[...]
    kvl = jnp.dot(ho, l2hwkv_ref[...], preferred_element_type=jnp.float32) + l2hbkv_ref[...]
    al = jax.nn.sigmoid(jnp.sum(ql2h * kvl[:, 0:D], axis=1, keepdims=True) * dscale)
    low_cross = al * kvl[:, D:2 * D]

    hn_ref[...] = _layer_norm(he + ho + high_cross, nhg_ref[...], nhb_ref[...]).reshape(1, CB, D)

    ri = lax.broadcasted_iota(jnp.int32, (R, CB), 0)
    rj = lax.broadcasted_iota(jnp.int32, (R, CB), 1)
    expand = jnp.where(ri // N_GENES == rj, 1.0, 0.0).astype(jnp.bfloat16)
    z = x + low_out + jnp.dot(expand, low_cross.astype(jnp.bfloat16),
                              preferred_element_type=jnp.float32)
    ln_ref[...] = _layer_norm(z, nlg_ref[...], nlb_ref[...]).reshape(CB, N_GENES, D)


def _full(shape):
    return pl.BlockSpec(shape, lambda i: tuple(0 for _ in shape))


_A_KWARGS = dict(
    grid=(1,),
    in_specs=[
        pl.BlockSpec((N_CELLS, D), lambda i: (0, 0)),
        pl.BlockSpec((NC, ACC_ROWS, D), lambda i: (0, 0, 0)),
        pl.BlockSpec((2, E_GRN), lambda i: (0, 0)),
        _full((D, 2 * D)), _full((1, 2 * D)),
        _full((2 * D, D)), _full((1, D)),
        _full((1, 1)), _full((1, D)), _full((1, D)),
    ],
    out_specs=[
        pl.BlockSpec((N_CELLS, D), lambda i: (0, 0)),
        pl.BlockSpec((R, R), lambda i: (0, 0)),
    ],
    out_shape=[
        jax.ShapeDtypeStruct((N_CELLS, D), jnp.float32),
        jax.ShapeDtypeStruct((R, R), jnp.float32),
    ],
)

_B_KWARGS = dict(
    grid=(N_CELLS // CB,),
    in_specs=[
        pl.BlockSpec((CB, N_GENES, D), lambda i: (i, 0, 0)),
        pl.BlockSpec((1, CB, D), lambda i: (i, 0, 0)),
        pl.BlockSpec((1, CB, D), lambda i: (i, 0, 0)),
        _full((R, R)),
        _full((D, 4 * D)), _full((1, 4 * D)),
        _full((1, D)), _full((1, D)),
        _full((D, D)), _full((1, D)),
        _full((D, D)), _full((1, D)),
        _full((D, 2 * D)), _full((1, 2 * D)),
        _full((D, D)), _full((1, D)),
        _full((D, 2 * D)), _full((1, 2 * D)),
        _full((1, D)), _full((1, D)), _full((1, D)), _full((1, D)),
    ],
    out_specs=[
        pl.BlockSpec((1, CB, D), lambda i: (i, 0, 0)),
        pl.BlockSpec((CB, N_GENES, D), lambda i: (i, 0, 0)),
    ],
    out_shape=[
        jax.ShapeDtypeStruct((N_CELLS // CB, CB, D), jnp.float32),
        jax.ShapeDtypeStruct((N_CELLS, N_GENES, D), jnp.float32),
    ],
)


def _row(b):
    return b.reshape(1, -1)


def kernel(high_emb, low_emb, spatial_edge_index, grn_edge_index, params):
    p = params
    agg_parts = _sc_agg(high_emb, spatial_edge_index.astype(jnp.int32))

    high_out, mtile = pl.pallas_call(_a_body, **_A_KWARGS)(
        high_emb, agg_parts, grn_edge_index.astype(jnp.int32),
        p["gin_W1"], _row(p["gin_b1"]), p["gin_W2"], _row(p["gin_b2"]),
        p["gin_eps"].reshape(1, 1), _row(p["gin_ln_g"]), _row(p["gin_ln_b"]),
    )

    s = 1.0 / math.sqrt(float(C))
    wqkvs = jnp.concatenate([p["tc_Wq"] * s, p["tc_Wk"], p["tc_Wv"], p["tc_Wskip"]],
                            axis=1).astype(jnp.bfloat16)
    bqkvs = jnp.concatenate([p["tc_bq"] * s, p["tc_bk"], p["tc_bv"], p["tc_bskip"]]).reshape(1, -1)
    h2lwkv = jnp.concatenate([p["h2l_Wk"], p["h2l_Wv"]], axis=1)
    h2lbkv = jnp.concatenate([p["h2l_bk"], p["h2l_bv"]]).reshape(1, -1)
    l2hwkv = jnp.concatenate([p["l2h_Wk"], p["l2h_Wv"]], axis=1)
    l2hbkv = jnp.concatenate([p["l2h_bk"], p["l2h_bv"]]).reshape(1, -1)

    high_new, low_new = pl.pallas_call(_b_body, **_B_KWARGS)(
        low_emb, high_emb.reshape(N_CELLS // CB, CB, D),
        high_out.reshape(N_CELLS // CB, CB, D), mtile,
        wqkvs, bqkvs, _row(p["tc_ln_g"]), _row(p["tc_ln_b"]),
        p["agg_W"], _row(p["agg_b"]),
        p["h2l_Wq"], _row(p["h2l_bq"]), h2lwkv, h2lbkv,
        p["l2h_Wq"], _row(p["l2h_bq"]), l2hwkv, l2hbkv,
        _row(p["nh_g"]), _row(p["nh_b"]), _row(p["nl_g"]), _row(p["nl_b"]),
    )
    return (high_new.reshape(N_CELLS, D), low_new)


# speedup vs baseline: 1.3674x; 1.1140x over previous
import functools
import math

import jax
import jax.numpy as jnp
from jax import lax
from jax.experimental import pallas as pl
from jax.experimental.pallas import tpu as pltpu
from jax.experimental.pallas import tpu_sc as plsc

D = 128
H = 4
C = 32
N_CELLS = 2000
N_GENES = 64
E_SPATIAL = 32000
E_GRN = 256

CB = 8
R = CB * N_GENES

NC = 2
NS = 16
NW = NC * NS
EPW = E_SPATIAL // NW
CHUNK = 128
NFULL = EPW // CHUNK
REM = EPW - NFULL * CHUNK
ACC_ROWS = 2048
OWN = ACC_ROWS // NS


def _sc_agg_body(x_hbm, src_hbm, dst_hbm, out_hbm,
                 rows_v, rows_rem_v, src_v, src_rem_v, dst_v, dst_rem_v,
                 zero_v, core_acc, sem):
    c = lax.axis_index("c")
    s = lax.axis_index("s")
    w = s * NC + c

    def _zr(i, _):
        for j in range(D // 16):
            zero_v[i, pl.ds(j * 16, 16)] = jnp.zeros((16,), jnp.float32)
        return 0
    lax.fori_loop(0, OWN, _zr, 0)

    pltpu.sync_copy(zero_v, core_acc.at[pl.ds(s * OWN, OWN)])
    plsc.subcore_barrier()

    base = w * EPW
    for j in range(NFULL):
        off = base + j * CHUNK
        pltpu.sync_copy(src_hbm.at[pl.ds(off, CHUNK)], src_v.at[0])
        pltpu.sync_copy(dst_hbm.at[pl.ds(off, CHUNK)], dst_v.at[0])
        pltpu.async_copy(x_hbm.at[src_v.at[0]], rows_v, sem).wait()
        pltpu.sync_copy(rows_v, core_acc.at[dst_v.at[0]], add=True)
    off = base + NFULL * CHUNK
    pltpu.sync_copy(src_hbm.at[pl.ds(off, REM)], src_rem_v.at[0])
    pltpu.sync_copy(dst_hbm.at[pl.ds(off, REM)], dst_rem_v.at[0])
    pltpu.async_copy(x_hbm.at[src_rem_v.at[0]], rows_rem_v, sem).wait()
    pltpu.sync_copy(rows_rem_v, core_acc.at[dst_rem_v.at[0]], add=True)

    plsc.subcore_barrier()
    pltpu.sync_copy(core_acc.at[pl.ds(s * OWN, OWN)], zero_v)
    pltpu.sync_copy(zero_v, out_hbm.at[c, pl.ds(s * OWN, OWN)])


def _sc_agg(high_emb, spatial_edge_index):
    mesh = plsc.VectorSubcoreMesh(core_axis_name="c", subcore_axis_name="s")
    k = functools.partial(
        pl.kernel,
        mesh=mesh,
        out_type=jax.ShapeDtypeStruct((NC, ACC_ROWS, D), jnp.float32),
        scratch_types=[
            pltpu.VMEM((CHUNK, D), jnp.float32),
            pltpu.VMEM((REM, D), jnp.float32),
            pltpu.VMEM((1, CHUNK), jnp.int32),
            pltpu.VMEM((1, REM), jnp.int32),
            pltpu.VMEM((1, CHUNK), jnp.int32),
            pltpu.VMEM((1, REM), jnp.int32),
            pltpu.VMEM((OWN, D), jnp.float32),
            pltpu.VMEM_SHARED((ACC_ROWS, D), jnp.float32),
            pltpu.SemaphoreType.DMA,
        ],
    )(_sc_agg_body)
    return k(high_emb, spatial_edge_index[0], spatial_edge_index[1])


def _layer_norm(x, g, b):
    m = jnp.mean(x, axis=-1, keepdims=True)
    v = jnp.mean((x - m) ** 2, axis=-1, keepdims=True)
    return (x - m) * lax.rsqrt(v + 1e-5) * g + b


def _gelu(x):
    return x * 0.5 * (1.0 + lax.erf(x * (2.0 ** -0.5)))


def _a_body(high_ref, agg_ref, grn_ref, w1_ref, b1_ref, w2_ref, b2_ref,
            eps_ref, g_ref, beta_ref, ho_ref, mt_ref):
    x = high_ref[...]
    agg = agg_ref[0, :N_CELLS, :] + agg_ref[1, :N_CELLS, :]
    h = (1.0 + eps_ref[0, 0]) * x + agg
    h = _gelu(jnp.dot(h, w1_ref[...], preferred_element_type=jnp.float32) + b1_ref[...])
    h = jnp.dot(h, w2_ref[...], preferred_element_type=jnp.float32) + b2_ref[...]
    ho_ref[...] = _layer_norm(h, g_ref[...], beta_ref[...])

    src = grn_ref[0:1, :]
    dst = grn_ref[1:2, :]
    gi = lax.broadcasted_iota(jnp.int32, (N_GENES, E_GRN), 0)
    dst_oh = (gi == dst).astype(jnp.float32)
    src_oh = (gi == src).astype(jnp.float32)
    m = lax.dot_general(dst_oh, src_oh, (((1,), (1,)), ((), ())),
                        preferred_element_type=jnp.float32)
    mrep = jnp.concatenate([jnp.concatenate([m] * CB, axis=1)] * CB, axis=0)
    rc = lax.broadcasted_iota(jnp.int32, (R, R), 0) // N_GENES
    cc = lax.broadcasted_iota(jnp.int32, (R, R), 1) // N_GENES
    mt = jnp.where(rc == cc, mrep, 0.0)
    mt_ref[...] = jnp.where(mt > 0.0, jnp.log(jnp.maximum(mt, 1e-30)), -1e30)


def _b_body(low_ref, he_ref, ho_ref, mt_ref, wqkvs_ref, bqkvs_ref,
            tcg_ref, tcb_ref, aggw_ref, aggb_ref,
            h2lwq_ref, h2lbq_ref, h2lwkv_ref, h2lbkv_ref,
            l2hwq_ref, l2hbq_ref, l2hwkv_ref, l2hbkv_ref,
            nhg_ref, nhb_ref, nlg_ref, nlb_ref,
            hn_ref, ln_ref):
    x = low_ref[...].reshape(R, D)
    qkvs = jnp.dot(x.astype(jnp.bfloat16), wqkvs_ref[...],
                   preferred_element_type=jnp.float32) + bqkvs_ref[...]
    q = qkvs[:, 0:D]
    k = qkvs[:, D:2 * D]
    v = qkvs[:, 2 * D:3 * D]
    skip = qkvs[:, 3 * D:4 * D]
    mt = mt_ref[...]

    lane_head = lax.broadcasted_iota(jnp.int32, (R, D), 1) // C
    od_sum = None
    for h in range(H):
        qh = q[:, h * C:(h + 1) * C].astype(jnp.bfloat16)
        kh = k[:, h * C:(h + 1) * C].astype(jnp.bfloat16)
        maskc = jnp.where(lane_head == h, 1.0, 0.0)
        rhs = jnp.concatenate([v * maskc, maskc], axis=1)
        logit = lax.dot_general(qh, kh, (((1,), (1,)), ((), ())),
                                preferred_element_type=jnp.float32)
        ex = jnp.exp(logit + mt)
        od = jnp.dot(ex, rhs, preferred_element_type=jnp.float32)
        od_sum = od if od_sum is None else od_sum + od
    denw = od_sum[:, D:2 * D]
    attn = od_sum[:, 0:D] / jnp.where(denw > 0.0, denw, 1.0) + skip
    low_out = _layer_norm(attn, tcg_ref[...], tcb_ref[...])

    ii = lax.broadcasted_iota(jnp.int32, (CB, R), 0)
    jj = lax.broadcasted_iota(jnp.int32, (CB, R), 1)
    pool = jnp.where(jj // N_GENES == ii, 1.0 / N_GENES, 0.0).astype(jnp.bfloat16)
    gmean = jnp.dot(pool, low_out.astype(jnp.bfloat16),
                    preferred_element_type=jnp.float32)
    gene = _gelu(jnp.dot(gmean, aggw_ref[...], preferred_element_type=jnp.float32) + aggb_ref[...])

    ho = ho_ref[0]
    he = he_ref[0]
    dscale = 1.0 / math.sqrt(float(D))
    qh2l = jnp.dot(ho, h2lwq_ref[...], preferred_element_type=jnp.float32) + h2lbq_ref[...]
    kvh = jnp.dot(gene, h2lwkv_ref[...], preferred_element_type=jnp.float32) + h2lbkv_ref[...]
    ah = jax.nn.sigmoid(jnp.sum(qh2l * kvh[:, 0:D], axis=1, keepdims=True) * dscale)
    high_cross = ah * kvh[:, D:2 * D]

    ql2h = jnp.dot(gene, l2hwq_ref[...], preferred_element_type=jnp.float32) + l2hbq_ref[...]
    kvl = jnp.dot(ho, l2hwkv_ref[...], preferred_element_type=jnp.float32) + l2hbkv_ref[...]
    al = jax.nn.sigmoid(jnp.sum(ql2h * kvl[:, 0:D], axis=1, keepdims=True) * dscale)
    low_cross = al * kvl[:, D:2 * D]

    hn_ref[...] = _layer_norm(he + ho + high_cross, nhg_ref[...], nhb_ref[...]).reshape(1, CB, D)

    ri = lax.broadcasted_iota(jnp.int32, (R, CB), 0)
    rj = lax.broadcasted_iota(jnp.int32, (R, CB), 1)
    expand = jnp.where(ri // N_GENES == rj, 1.0, 0.0).astype(jnp.bfloat16)
    z = x + low_out + jnp.dot(expand, low_cross.astype(jnp.bfloat16),
                              preferred_element_type=jnp.float32)
    ln_ref[...] = _layer_norm(z, nlg_ref[...], nlb_ref[...]).reshape(CB, N_GENES, D)


def _full(shape):
    return pl.BlockSpec(shape, lambda i: tuple(0 for _ in shape))


_A_KWARGS = dict(
    grid=(1,),
    in_specs=[
        pl.BlockSpec((N_CELLS, D), lambda i: (0, 0)),
        pl.BlockSpec((NC, ACC_ROWS, D), lambda i: (0, 0, 0)),
        pl.BlockSpec((2, E_GRN), lambda i: (0, 0)),
        _full((D, 2 * D)), _full((1, 2 * D)),
        _full((2 * D, D)), _full((1, D)),
        _full((1, 1)), _full((1, D)), _full((1, D)),
    ],
    out_specs=[
        pl.BlockSpec((N_CELLS, D), lambda i: (0, 0)),
        pl.BlockSpec((R, R), lambda i: (0, 0)),
    ],
    out_shape=[
        jax.ShapeDtypeStruct((N_CELLS, D), jnp.float32),
        jax.ShapeDtypeStruct((R, R), jnp.float32),
    ],
)

_B_KWARGS = dict(
    grid=(N_CELLS // CB,),
    in_specs=[
        pl.BlockSpec((CB, N_GENES, D), lambda i: (i, 0, 0)),
        pl.BlockSpec((1, CB, D), lambda i: (i, 0, 0)),
        pl.BlockSpec((1, CB, D), lambda i: (i, 0, 0)),
        _full((R, R)),
        _full((D, 4 * D)), _full((1, 4 * D)),
        _full((1, D)), _full((1, D)),
        _full((D, D)), _full((1, D)),
        _full((D, D)), _full((1, D)),
        _full((D, 2 * D)), _full((1, 2 * D)),
        _full((D, D)), _full((1, D)),
        _full((D, 2 * D)), _full((1, 2 * D)),
        _full((1, D)), _full((1, D)), _full((1, D)), _full((1, D)),
    ],
    out_specs=[
        pl.BlockSpec((1, CB, D), lambda i: (i, 0, 0)),
        pl.BlockSpec((CB, N_GENES, D), lambda i: (i, 0, 0)),
    ],
    out_shape=[
        jax.ShapeDtypeStruct((N_CELLS // CB, CB, D), jnp.float32),
        jax.ShapeDtypeStruct((N_CELLS, N_GENES, D), jnp.float32),
    ],
)


def _row(b):
    return b.reshape(1, -1)


def kernel(high_emb, low_emb, spatial_edge_index, grn_edge_index, params):
    p = params
    agg_parts = _sc_agg(high_emb, spatial_edge_index.astype(jnp.int32))

    high_out, mtile = pl.pallas_call(_a_body, **_A_KWARGS)(
        high_emb, agg_parts, grn_edge_index.astype(jnp.int32),
        p["gin_W1"], _row(p["gin_b1"]), p["gin_W2"], _row(p["gin_b2"]),
        p["gin_eps"].reshape(1, 1), _row(p["gin_ln_g"]), _row(p["gin_ln_b"]),
    )

    s = 1.0 / math.sqrt(float(C))
    wqkvs = jnp.concatenate([p["tc_Wq"] * s, p["tc_Wk"], p["tc_Wv"], p["tc_Wskip"]],
                            axis=1).astype(jnp.bfloat16)
    bqkvs = jnp.concatenate([p["tc_bq"] * s, p["tc_bk"], p["tc_bv"], p["tc_bskip"]]).reshape(1, -1)
    h2lwkv = jnp.concatenate([p["h2l_Wk"], p["h2l_Wv"]], axis=1)
    h2lbkv = jnp.concatenate([p["h2l_bk"], p["h2l_bv"]]).reshape(1, -1)
    l2hwkv = jnp.concatenate([p["l2h_Wk"], p["l2h_Wv"]], axis=1)
    l2hbkv = jnp.concatenate([p["l2h_bk"], p["l2h_bv"]]).reshape(1, -1)

    high_new, low_new = pl.pallas_call(_b_body, **_B_KWARGS)(
        low_emb, high_emb.reshape(N_CELLS // CB, CB, D),
        high_out.reshape(N_CELLS // CB, CB, D), mtile,
        wqkvs, bqkvs, _row(p["tc_ln_g"]), _row(p["tc_ln_b"]),
        p["agg_W"], _row(p["agg_b"]),
        p["h2l_Wq"], _row(p["h2l_bq"]), h2lwkv, h2lbkv,
        p["l2h_Wq"], _row(p["l2h_bq"]), l2hwkv, l2hbkv,
        _row(p["nh_g"]), _row(p["nh_b"]), _row(p["nl_g"]), _row(p["nl_b"]),
    )
    return (high_new.reshape(N_CELLS, D), low_new)
